# baseline (speedup 1.0000x reference)
import functools

import jax
import jax.numpy as jnp
from jax import lax
from jax.experimental import pallas as pl
from jax.experimental.pallas import tpu as pltpu
from jax.experimental.pallas import tpu_sc as plsc

N = 10000
E = 160000
IN_FEAT = 256
HID = 256
HC = 512
OUT = 40
BETA = 0.5
EPS = 1e-5

NPAD = 10016
E2 = 163840
EPW_A = E2 // 32
KA = 128
EPT_B = E2 // 16
KB = 128
RPT = NPAD // 16


def _ln(x, g, b):
    mu = x.mean(axis=-1, keepdims=True)
    var = ((x - mu) ** 2).mean(axis=-1, keepdims=True)
    return (x - mu) / jnp.sqrt(var + EPS) * g + b


def _dense0_body(x_ref, w_ref, b_ref, g_ref, bb_ref, o_ref):
    h = jnp.dot(x_ref[...], w_ref[...], preferred_element_type=jnp.float32)
    h = jax.nn.relu(h + b_ref[...])
    o_ref[...] = _ln(h, g_ref[...], bb_ref[...])


def _dense0(x, W1, b1, g, bb):
    BM = 1000
    return pl.pallas_call(
        _dense0_body,
        grid=(N // BM,),
        in_specs=[
            pl.BlockSpec((BM, IN_FEAT), lambda i: (i, 0)),
            pl.BlockSpec((IN_FEAT, HC), lambda i: (0, 0)),
            pl.BlockSpec((HC,), lambda i: (0,)),
            pl.BlockSpec((HC,), lambda i: (0,)),
            pl.BlockSpec((HC,), lambda i: (0,)),
        ],
        out_specs=pl.BlockSpec((BM, HC), lambda i: (i, 0)),
        out_shape=jax.ShapeDtypeStruct((N, HC), jnp.float32),
    )(x, W1, b1, g, bb)


def _mm_body(a_ref, w_ref, o_ref, oh_ref, o4_ref):
    o = jnp.dot(a_ref[...], w_ref[...], preferred_element_type=jnp.float32)
    o_ref[...] = o
    oh_ref[...] = 0.5 * o
    for g in range(4):
        o4_ref[g] = o[:, g * 64:(g + 1) * 64]


def _mm(h, w):
    BM = 1000
    return pl.pallas_call(
        _mm_body,
        grid=(N // BM,),
        in_specs=[
            pl.BlockSpec((BM, HC), lambda i: (i, 0)),
            pl.BlockSpec((HC, HID), lambda i: (0, 0)),
        ],
        out_specs=[
            pl.BlockSpec((BM, HID), lambda i: (i, 0)),
            pl.BlockSpec((BM, HID), lambda i: (i, 0)),
            pl.BlockSpec((4, BM, 64), lambda i: (0, i, 0)),
        ],
        out_shape=[
            jax.ShapeDtypeStruct((N, HID), jnp.float32),
            jax.ShapeDtypeStruct((N, HID), jnp.float32),
            jax.ShapeDtypeStruct((4, N, 64), jnp.float32),
        ],
    )(h, w)


def _post_body(s0_ref, sa_ref, ego_ref, g_ref, b_ref, o_ref):
    s0 = s0_ref[...]
    seg = jnp.concatenate([s0, sa_ref[...] - s0], axis=-1)
    h2 = _ln(jax.nn.relu(seg), g_ref[...], b_ref[...])
    o_ref[...] = (1.0 - BETA) * h2 + BETA * ego_ref[...]


def _post(seg0, segA, ego, g, b):
    BM = 1000
    return pl.pallas_call(
        _post_body,
        grid=(N // BM,),
        in_specs=[
            pl.BlockSpec((BM, HID), lambda i: (i, 0)),
            pl.BlockSpec((BM, HID), lambda i: (i, 0)),
            pl.BlockSpec((BM, HC), lambda i: (i, 0)),
            pl.BlockSpec((HC,), lambda i: (0,)),
            pl.BlockSpec((HC,), lambda i: (0,)),
        ],
        out_specs=pl.BlockSpec((BM, HC), lambda i: (i, 0)),
        out_shape=jax.ShapeDtypeStruct((N, HC), jnp.float32),
    )(seg0, segA, ego, g, b)


def _final_body(h_ref, w_ref, b_ref, o_ref):
    o = jnp.dot(h_ref[...], w_ref[...], preferred_element_type=jnp.float32) + b_ref[...]
    o_ref[...] = jax.nn.log_softmax(o, axis=-1)


def _final(h, W2, b2):
    BM = 1000
    return pl.pallas_call(
        _final_body,
        grid=(N // BM,),
        in_specs=[
            pl.BlockSpec((BM, HC), lambda i: (i, 0)),
            pl.BlockSpec((HC, OUT), lambda i: (0, 0)),
            pl.BlockSpec((OUT,), lambda i: (0,)),
        ],
        out_specs=pl.BlockSpec((BM, OUT), lambda i: (i, 0)),
        out_shape=jax.ShapeDtypeStruct((N, OUT), jnp.float32),
    )(h, W2, b2)


def _att_body(hph_hbm, hp_hbm, row_hbm, col_hbm, wa_hbm, att_hbm,
              rbuf, cbuf, ridx, cidx, abuf, wavm, sem_r, sem_c):
    c = lax.axis_index("c")
    s = lax.axis_index("s")
    wid = s * 2 + c
    pltpu.sync_copy(wa_hbm, wavm)
    ev0 = lax.iota(jnp.int32, 16)
    base0 = wid * EPW_A

    def chunk_body(i, carry):
        base = base0 + i * KA
        pltpu.sync_copy(row_hbm.at[pl.ds(base, KA)], ridx)
        pltpu.sync_copy(col_hbm.at[pl.ds(base, KA)], cidx)
        cp_r = pltpu.async_copy(hph_hbm.at[ridx], rbuf, sem_r)
        cp_c = pltpu.async_copy(hp_hbm.at[cidx], cbuf, sem_c)
        cp_r.wait()
        cp_c.wait()
        for g in range(KA // 16):
            ev = ev0 + (g * 16)

            def dot_body(jb, pc):
                p0, p1 = pc
                jb16 = jb * 16
                wv0 = wavm[0, pl.ds(jb16, 16)]
                wv1 = wavm[1, pl.ds(jb16, 16)]
                dv0 = jnp.full((16,), 0, dtype=jnp.int32) + jb16
                for k in range(16):
                    rv = plsc.load_gather(rbuf, [ev, dv0 + k])
                    cv = plsc.load_gather(cbuf, [ev, dv0 + k])
                    t = jnp.maximum(rv + cv, 0.0)
                    p0 = p0 + t * _lane_bcast(wv0, k)
                    p1 = p1 + t * _lane_bcast(wv1, k)
                return (p0, p1)

            z = jnp.zeros((16,), jnp.float32)
            p0, p1 = lax.fori_loop(0, HID // 16, dot_body, (z, z))
            a = 1.0 / (1.0 + jnp.exp(p1 - p0))
            abuf[pl.ds(g * 16, 16)] = a
        pltpu.sync_copy(abuf, att_hbm.at[pl.ds(base, KA)])
        return carry

    lax.fori_loop(0, EPW_A // KA, chunk_body, 0)


def _phase_a(hph_pad, hp_pad, row_p, col_p, waT):
    mesh = plsc.VectorSubcoreMesh(core_axis_name="c", subcore_axis_name="s")
    f = functools.partial(
        pl.kernel,
        out_type=jax.ShapeDtypeStruct((E2,), jnp.float32),
        mesh=mesh,
        compiler_params=pltpu.CompilerParams(use_tc_tiling_on_sc=False, needs_layout_passes=False),
        scratch_types=[
            pltpu.VMEM((KA, HID), jnp.float32),
            pltpu.VMEM((KA, HID), jnp.float32),
            pltpu.VMEM((KA,), jnp.int32),
            pltpu.VMEM((KA,), jnp.int32),
            pltpu.VMEM((KA,), jnp.float32),
            pltpu.VMEM((2, HID), jnp.float32),
            pltpu.SemaphoreType.DMA,
            pltpu.SemaphoreType.DMA,
        ],
    )(_att_body)
    return f(hph_pad, hp_pad, row_p, col_p, waT)


def _lane_bcast(v, u):
    idx = jnp.full((16, 1), u, dtype=jnp.int32)
    dnums = lax.GatherDimensionNumbers(
        offset_dims=(), collapsed_slice_dims=(0,), start_index_map=(0,))
    return lax.gather(v, idx, dnums, (1,),
                      mode=lax.GatherScatterMode.PROMISE_IN_BOUNDS)


def _agg_body(hp4f_hbm, row_hbm, col_hbm, att_hbm, zeros_hbm,
              out0_hbm, outA_hbm,
              acc0, accA, cbuf, sbuf, ridx, cidx, cidx2, abuf,
              sem_g, sem_s0, sem_sA):
    c = lax.axis_index("c")
    s = lax.axis_index("s")
    rows0 = s * RPT
    for lg in range(2):
        g = c * 2 + lg
        goff = g * NPAD
        pltpu.sync_copy(zeros_hbm.at[pl.ds(rows0, RPT)], acc0.at[pl.ds(rows0, RPT)])
        pltpu.sync_copy(zeros_hbm.at[pl.ds(rows0, RPT)], accA.at[pl.ds(rows0, RPT)])
        plsc.subcore_barrier()

        def chunk_body(i, carry):
            base = s * EPT_B + i * KB
            pltpu.sync_copy(row_hbm.at[pl.ds(base, KB)], ridx)
            pltpu.sync_copy(col_hbm.at[pl.ds(base, KB)], cidx)
            pltpu.sync_copy(att_hbm.at[pl.ds(base, KB)], abuf)
            for q in range(KB // 16):
                cidx2[pl.ds(q * 16, 16)] = cidx[pl.ds(q * 16, 16)] + goff
            pltpu.async_copy(hp4f_hbm.at[cidx2], cbuf, sem_g).wait()
            for q in range(KB // 16):
                av = abuf[pl.ds(q * 16, 16)]
                for u in range(16):
                    e = q * 16 + u
                    a_bc = _lane_bcast(av, u)
                    for j in range(4):
                        sbuf[e, pl.ds(j * 16, 16)] = (
                            cbuf[e, pl.ds(j * 16, 16)] * a_bc)
            cpA = pltpu.async_copy(cbuf, accA.at[ridx], sem_sA, add=True)
            cp0 = pltpu.async_copy(sbuf, acc0.at[ridx], sem_s0, add=True)
            cpA.wait()
            cp0.wait()
            return carry

        lax.fori_loop(0, EPT_B // KB, chunk_body, 0)
        plsc.subcore_barrier()
        pltpu.sync_copy(acc0.at[pl.ds(rows0, RPT)],
                        out0_hbm.at[pl.ds(goff + rows0, RPT)])
        pltpu.sync_copy(accA.at[pl.ds(rows0, RPT)],
                        outA_hbm.at[pl.ds(goff + rows0, RPT)])
        plsc.subcore_barrier()


def _phase_b(hp4f, row_p, col_p, att, zeros_rows):
    mesh = plsc.VectorSubcoreMesh(core_axis_name="c", subcore_axis_name="s")
    f = functools.partial(
        pl.kernel,
        out_type=[
            jax.ShapeDtypeStruct((4 * NPAD, 64), jnp.float32),
            jax.ShapeDtypeStruct((4 * NPAD, 64), jnp.float32),
        ],
        mesh=mesh,
        compiler_params=pltpu.CompilerParams(use_tc_tiling_on_sc=False, needs_layout_passes=False),
        scratch_types=[
            pltpu.VMEM_SHARED((NPAD, 64), jnp.float32),
            pltpu.VMEM_SHARED((NPAD, 64), jnp.float32),
            pltpu.VMEM((KB, 64), jnp.float32),
            pltpu.VMEM((KB, 64), jnp.float32),
            pltpu.VMEM((KB,), jnp.int32),
            pltpu.VMEM((KB,), jnp.int32),
            pltpu.VMEM((KB,), jnp.int32),
            pltpu.VMEM((KB,), jnp.float32),
            pltpu.SemaphoreType.DMA,
            pltpu.SemaphoreType.DMA,
            pltpu.SemaphoreType.DMA,
        ],
    )(_agg_body)
    return f(hp4f, row_p, col_p, att, zeros_rows)


def _edge_layer(h, row_p, col_p, waT, w_lin, zeros_rows):
    hp, hph, hp4 = _mm(h, w_lin)
    hp_pad = jnp.pad(hp, ((0, NPAD - N), (0, 0)))
    hph_pad = jnp.pad(hph, ((0, NPAD - N), (0, 0)))
    hp4f = jnp.pad(hp4, ((0, 0), (0, NPAD - N), (0, 0))).reshape(4 * NPAD, 64)
    att = _phase_a(hph_pad, hp_pad, row_p, col_p, waT)
    out0f, outAf = _phase_b(hp4f, row_p, col_p, att, zeros_rows)
    seg0 = out0f.reshape(4, NPAD, 64)[:, :N, :].transpose(1, 0, 2).reshape(N, HID)
    segA = outAf.reshape(4, NPAD, 64)[:, :N, :].transpose(1, 0, 2).reshape(N, HID)
    return seg0, segA


def kernel(x, edge_index, W1, b1, ln0_g, ln0_b, lin_w0, att_w0, ln1_g, ln1_b,
           lin_w1, att_w1, ln2_g, ln2_b, W2, b2):
    row = edge_index[0]
    col = edge_index[1]
    row = jnp.where(row != col, row, N)
    row_p = jnp.concatenate([row, jnp.full((E2 - E,), N, jnp.int32)])
    col_p = jnp.concatenate([col, jnp.zeros((E2 - E,), jnp.int32)])
    zeros_rows = jnp.zeros((NPAD, 64), jnp.float32)

    h = _dense0(x, W1, b1, ln0_g, ln0_b)
    ego = h
    for (wl, wa, g, bb) in ((lin_w0, att_w0, ln1_g, ln1_b),
                            (lin_w1, att_w1, ln2_g, ln2_b)):
        seg0, segA = _edge_layer(h, row_p, col_p, wa.T, wl, zeros_rows)
        h = _post(seg0, segA, ego, g, bb)
    return _final(h, W2, b2)
